# baseline (speedup 1.0000x reference)
import functools

import jax
import jax.numpy as jnp
from jax import lax
from jax.experimental import pallas as pl
from jax.experimental.pallas import tpu as pltpu
from jax.experimental.pallas import tpu_sc as plsc

B = 16384
D = 64
NC = 2
NS = 16
NW = NC * NS
BPW = B // NW
CH = 128
NCH = BPW // CH

TC_ROWS = 2048


def _sc_body(gu_hbm, gi_hbm, user_hbm, item_hbm,
             gou_hbm, goi_hbm,
             idx_u, idx_i, gu_v, gi_v,
             sem_idx, sem_gat, sem_out):
    wid = lax.axis_index("s") * NC + lax.axis_index("c")
    base = wid * BPW

    cu = pltpu.async_copy(user_hbm.at[wid], idx_u, sem_idx)
    ci = pltpu.async_copy(item_hbm.at[wid], idx_i, sem_idx)
    cu.wait()
    ci.wait()

    gathers = []
    for j in range(NCH):
        gathers.append(pltpu.async_copy(
            gu_hbm.at[idx_u.at[j]], gu_v.at[pl.ds(j * CH, CH)], sem_gat))
        gathers.append(pltpu.async_copy(
            gi_hbm.at[idx_i.at[j]], gi_v.at[pl.ds(j * CH, CH)], sem_gat))
    for c in gathers:
        c.wait()

    ou = pltpu.async_copy(gu_v, gou_hbm.at[pl.ds(base, BPW)], sem_out)
    oi = pltpu.async_copy(gi_v, goi_hbm.at[pl.ds(base, BPW)], sem_out)
    ou.wait()
    oi.wait()


def _tc_body(gu_ref, gi_ref, out_ref):
    out_ref[...] = jnp.sum(gu_ref[...] * gi_ref[...], axis=1)


@jax.jit
def _run(Gu, Gi, user_r, item_r):
    mesh = plsc.VectorSubcoreMesh(core_axis_name="c", subcore_axis_name="s")
    gather_fn = pl.kernel(
        _sc_body,
        out_type=[
            jax.ShapeDtypeStruct((B, D), jnp.float32),
            jax.ShapeDtypeStruct((B, D), jnp.float32),
        ],
        mesh=mesh,
        compiler_params=pltpu.CompilerParams(use_tc_tiling_on_sc=False),
        scratch_types=[
            pltpu.VMEM((NCH, CH), jnp.int32),
            pltpu.VMEM((NCH, CH), jnp.int32),
            pltpu.VMEM((BPW, D), jnp.float32),
            pltpu.VMEM((BPW, D), jnp.float32),
            pltpu.SemaphoreType.DMA,
            pltpu.SemaphoreType.DMA,
            pltpu.SemaphoreType.DMA,
        ],
    )
    gamma_u, gamma_i = gather_fn(Gu, Gi, user_r, item_r)

    xui = pl.pallas_call(
        _tc_body,
        grid=(B // TC_ROWS,),
        in_specs=[
            pl.BlockSpec((TC_ROWS, D), lambda i: (i, 0)),
            pl.BlockSpec((TC_ROWS, D), lambda i: (i, 0)),
        ],
        out_specs=pl.BlockSpec((TC_ROWS,), lambda i: (i,)),
        out_shape=jax.ShapeDtypeStruct((B,), jnp.float32),
    )(gamma_u, gamma_i)

    return xui, gamma_u, gamma_i


def kernel(Gu, Gi, user, item):
    user_r = user.astype(jnp.int32).reshape(NW, NCH, CH)
    item_r = item.astype(jnp.int32).reshape(NW, NCH, CH)
    xui, gamma_u, gamma_i = _run(Gu, Gi, user_r, item_r)
    return (xui, gamma_u, gamma_i)
